# baseline (speedup 1.0000x reference)
import functools

import jax
import jax.numpy as jnp
from jax import lax
from jax.experimental import pallas as pl
from jax.experimental.pallas import tpu as pltpu
from jax.experimental.pallas import tpu_sc as plsc

S = 2048
H = 768
E = 8
TOPK = 2
F = 3072
EP = 128
TB = 512
BLK = 128
NBLK = 40
CAP = NBLK * BLK
NW = 32


def _router_body(x_ref, gw_ref, i1_ref, i2_ref, p1_ref, p2_ref, aux_ref,
                 cnt_acc, sp_acc):
    i = pl.program_id(0)
    nsteps = pl.num_programs(0)
    logits = lax.dot_general(
        x_ref[...].astype(jnp.bfloat16), gw_ref[...].astype(jnp.bfloat16),
        (((1,), (1,)), ((), ())),
        preferred_element_type=jnp.float32)
    col = lax.broadcasted_iota(jnp.int32, (TB, EP), 1)
    neg = jnp.float32(-1e30)
    lg = jnp.where(col < E, logits, neg)
    m = jnp.max(lg, axis=1, keepdims=True)
    ex = jnp.where(col < E, jnp.exp(lg - m), 0.0)
    z = jnp.sum(ex, axis=1, keepdims=True)
    probs = ex / z
    pm = jnp.where(col < E, probs, neg)
    p1 = jnp.max(pm, axis=1, keepdims=True)
    i1 = jnp.min(jnp.where(pm == p1, col, EP), axis=1, keepdims=True)
    pm2 = jnp.where(col == i1, neg, pm)
    p2 = jnp.max(pm2, axis=1, keepdims=True)
    i2 = jnp.min(jnp.where(pm2 == p2, col, EP), axis=1, keepdims=True)
    ps = p1 + p2
    i1_ref[...] = i1[:, 0]
    i2_ref[...] = i2[:, 0]
    p1_ref[...] = (p1 / ps)[:, 0]
    p2_ref[...] = (p2 / ps)[:, 0]

    oh = (col == i1).astype(jnp.float32) + (col == i2).astype(jnp.float32)
    cnt_blk = jnp.sum(oh, axis=0, keepdims=True)
    sp_blk = jnp.sum(probs, axis=0, keepdims=True)

    @pl.when(i == 0)
    def _():
        cnt_acc[...] = jnp.zeros_like(cnt_acc)
        sp_acc[...] = jnp.zeros_like(sp_acc)

    cnt_acc[...] += cnt_blk
    sp_acc[...] += sp_blk

    @pl.when(i == nsteps - 1)
    def _():
        aux_ref[...] = ((jnp.float32(E) / jnp.float32(S)) * jnp.sum(
            cnt_acc[...] * sp_acc[...])).reshape(1, 1)


def _router(x_flat, gw_pad):
    return pl.pallas_call(
        _router_body,
        grid=(S // TB,),
        in_specs=[
            pl.BlockSpec((TB, H), lambda i: (i, 0)),
            pl.BlockSpec((EP, H), lambda i: (0, 0)),
        ],
        out_specs=[
            pl.BlockSpec((TB,), lambda i: (i,)),
            pl.BlockSpec((TB,), lambda i: (i,)),
            pl.BlockSpec((TB,), lambda i: (i,)),
            pl.BlockSpec((TB,), lambda i: (i,)),
            pl.BlockSpec((1, 1), lambda i: (0, 0)),
        ],
        out_shape=[
            jax.ShapeDtypeStruct((S,), jnp.int32),
            jax.ShapeDtypeStruct((S,), jnp.int32),
            jax.ShapeDtypeStruct((S,), jnp.float32),
            jax.ShapeDtypeStruct((S,), jnp.float32),
            jax.ShapeDtypeStruct((1, 1), jnp.float32),
        ],
        scratch_shapes=[
            pltpu.VMEM((1, EP), jnp.float32),
            pltpu.VMEM((1, EP), jnp.float32),
        ],
    )(x_flat, gw_pad)


FS = F // 2


def _ffn_body(bmeta_ref, xg_ref, w1_ref, b1_ref, w2_ref, b2_ref, out_ref):
    i = pl.program_id(0)
    j = pl.program_id(1)

    @pl.when(bmeta_ref[1, i] == 1)
    def _():
        xb = xg_ref[...].astype(jnp.bfloat16)
        h = lax.dot_general(
            xb, w1_ref[0].astype(jnp.bfloat16), (((1,), (1,)), ((), ())),
            preferred_element_type=jnp.float32)
        h = h + b1_ref[0]
        g = 0.5 * h * (1.0 + lax.erf(h * jnp.float32(0.7071067811865476)))
        y = lax.dot_general(
            g.astype(jnp.bfloat16), w2_ref[0].astype(jnp.bfloat16),
            (((1,), (1,)), ((), ())),
            preferred_element_type=jnp.float32)

        @pl.when(j == 0)
        def _():
            out_ref[...] = y + b2_ref[0]

        @pl.when(j == 1)
        def _():
            out_ref[...] += y


def _ffn(bmeta, xg, fc1_w, fc1_b, fc2_w, fc2_b):
    grid_spec = pltpu.PrefetchScalarGridSpec(
        num_scalar_prefetch=1,
        grid=(NBLK, 2),
        in_specs=[
            pl.BlockSpec((BLK, H), lambda i, j, m: (i, 0)),
            pl.BlockSpec((1, FS, H), lambda i, j, m: (m[0, i], j, 0)),
            pl.BlockSpec((1, 1, FS), lambda i, j, m: (m[0, i], 0, j)),
            pl.BlockSpec((1, H, FS), lambda i, j, m: (m[0, i], 0, j)),
            pl.BlockSpec((1, 1, H), lambda i, j, m: (m[0, i], 0, 0)),
        ],
        out_specs=pl.BlockSpec((BLK, H), lambda i, j, m: (i, 0)),
    )
    return pl.pallas_call(
        _ffn_body,
        grid_spec=grid_spec,
        out_shape=jax.ShapeDtypeStruct((CAP, H), jnp.float32),
    )(bmeta, xg, fc1_w, fc1_b, fc2_w, fc2_b)


_TW2 = S // NW


def _sc_dispatch(x_flat, pos0, pos1):
    mesh = plsc.VectorSubcoreMesh(core_axis_name="c", subcore_axis_name="s")
    p2d0 = pos0.reshape(NW, _TW2)
    p2d1 = pos1.reshape(NW, _TW2)

    @functools.partial(
        pl.kernel, mesh=mesh,
        out_type=jax.ShapeDtypeStruct((CAP, H), jnp.float32),
        scratch_types=[
            pltpu.VMEM((_TW2,), jnp.int32),
            pltpu.VMEM((_TW2,), jnp.int32),
            pltpu.VMEM((_TW2, H), jnp.float32),
            pltpu.SemaphoreType.DMA,
            pltpu.SemaphoreType.DMA,
        ],
    )
    def k2(x_hbm, p0_hbm, p1_hbm, xg_hbm, i0_v, i1_v, rows_v, sem0, sem1):
        wid = lax.axis_index("s") * 2 + lax.axis_index("c")
        j0 = pltpu.async_copy(p0_hbm.at[wid], i0_v, sem0)
        j1 = pltpu.async_copy(p1_hbm.at[wid], i1_v, sem1)
        r = pltpu.async_copy(x_hbm.at[pl.ds(wid * _TW2, _TW2)], rows_v, sem0)
        j0.wait()
        j1.wait()
        r.wait()
        w0 = pltpu.async_copy(rows_v, xg_hbm.at[i0_v], sem0)
        w1 = pltpu.async_copy(rows_v, xg_hbm.at[i1_v], sem1)
        w0.wait()
        w1.wait()

    return k2(x_flat, p2d0, p2d1)


_TW = S // NW


def _sc_combine(yg, pos0, pos1, pb0):
    mesh = plsc.VectorSubcoreMesh(core_axis_name="c", subcore_axis_name="s")
    p2d0 = pos0.reshape(NW, _TW)
    p2d1 = pos1.reshape(NW, _TW)
    pb3d = pb0.reshape(NW, _TW, 16)

    @functools.partial(
        pl.kernel, mesh=mesh,
        out_type=jax.ShapeDtypeStruct((S, H), jnp.float32),
        scratch_types=[
            pltpu.VMEM((_TW,), jnp.int32),
            pltpu.VMEM((_TW,), jnp.int32),
            pltpu.VMEM((_TW, 16), jnp.float32),
            pltpu.VMEM((_TW, H), jnp.float32),
            pltpu.VMEM((_TW, H), jnp.float32),
            pltpu.SemaphoreType.DMA,
            pltpu.SemaphoreType.DMA,
        ],
    )
    def k4(yg_hbm, p0_hbm, p1_hbm, pb_hbm, o_hbm, i0_v, i1_v, pb_v, acc_v,
           tmp_v, sem, sem1):
        wid = lax.axis_index("s") * 2 + lax.axis_index("c")
        j0 = pltpu.async_copy(p0_hbm.at[wid], i0_v, sem)
        j1 = pltpu.async_copy(p1_hbm.at[wid], i1_v, sem1)
        jp = pltpu.async_copy(pb_hbm.at[wid], pb_v, sem)
        j0.wait()
        jp.wait()
        g0 = pltpu.async_copy(yg_hbm.at[i0_v], acc_v, sem)
        j1.wait()
        g1 = pltpu.async_copy(yg_hbm.at[i1_v], tmp_v, sem1)
        g0.wait()
        g1.wait()

        @pl.loop(0, _TW)
        def _(r):
            p = pb_v[r, :]
            for c in range(H // 16):
                sl = pl.ds(c * 16, 16)
                y1 = tmp_v[r, sl]
                acc_v[r, sl] = y1 + p * (acc_v[r, sl] - y1)

        pltpu.sync_copy(acc_v, o_hbm.at[pl.ds(wid * _TW, _TW)])

    return k4(yg, p2d0, p2d1, pb3d)


def kernel(x, gate_w, fc1_w, fc1_b, fc2_w, fc2_b):
    b, s, h = x.shape
    x_flat = x.reshape(s, h)
    gw_pad = jnp.zeros((EP, h), jnp.float32).at[:E].set(gate_w)

    i1, i2, p1n, p2n, aux = _router(x_flat, gw_pad)

    e_flat = jnp.stack([i1, i2], axis=1).reshape(-1)
    oh = (e_flat[:, None] == jnp.arange(E, dtype=jnp.int32)[None, :])
    csum = jnp.cumsum(oh.astype(jnp.int32), axis=0)
    counts = csum[-1]
    rank = jnp.take_along_axis(csum, e_flat[:, None], axis=1)[:, 0] - 1
    pc = ((counts + BLK - 1) // BLK) * BLK
    pstart = jnp.cumsum(pc) - pc
    pos = jnp.take(pstart, e_flat) + rank
    pos0, pos1 = pos[0::2], pos[1::2]
    pb0 = jnp.broadcast_to(p1n[:, None], (S, 16))
    nvb = jnp.sum(pc) // BLK
    bidx = jnp.arange(NBLK, dtype=jnp.int32)
    gend = (pstart + pc) // BLK
    bexp_raw = jnp.sum((bidx[:, None] >= gend[None, :]).astype(jnp.int32),
                       axis=1)
    bvalid = bidx < nvb
    last_exp = jnp.max(jnp.where(bvalid, bexp_raw, 0))
    bexp = jnp.where(bvalid, bexp_raw, last_exp)
    bnew = jnp.concatenate(
        [jnp.ones((1,), jnp.int32),
         (bexp[1:] != bexp[:-1]).astype(jnp.int32)])
    bmeta = jnp.stack([bexp, bvalid.astype(jnp.int32), bnew], axis=0)

    xg = _sc_dispatch(x_flat, pos0, pos1)
    yg = _ffn(bmeta, xg, fc1_w, fc1_b.reshape(E, 1, F),
              fc2_w, fc2_b.reshape(E, 1, H))
    out = _sc_combine(yg, pos0, pos1, pb0)
    return out.reshape(b, s, h), aux[0, 0]


# speedup vs baseline: 1.3351x; 1.3351x over previous
import functools

import jax
import jax.numpy as jnp
from jax import lax
from jax.experimental import pallas as pl
from jax.experimental.pallas import tpu as pltpu
from jax.experimental.pallas import tpu_sc as plsc

S = 2048
H = 768
E = 8
TOPK = 2
F = 3072
EP = 128
TB = 512
BLK = 128
NBLK = 40
CAP = NBLK * BLK
NW = 32


def _router_body(x_ref, gw_ref, i1_ref, i2_ref, p1_ref, p2_ref, aux_ref,
                 cnt_acc, sp_acc):
    i = pl.program_id(0)
    nsteps = pl.num_programs(0)
    logits = lax.dot_general(
        x_ref[...].astype(jnp.bfloat16), gw_ref[...].astype(jnp.bfloat16),
        (((1,), (1,)), ((), ())),
        preferred_element_type=jnp.float32)
    col = lax.broadcasted_iota(jnp.int32, (TB, EP), 1)
    neg = jnp.float32(-1e30)
    lg = jnp.where(col < E, logits, neg)
    m = jnp.max(lg, axis=1, keepdims=True)
    ex = jnp.where(col < E, jnp.exp(lg - m), 0.0)
    z = jnp.sum(ex, axis=1, keepdims=True)
    probs = ex / z
    pm = jnp.where(col < E, probs, neg)
    p1 = jnp.max(pm, axis=1, keepdims=True)
    i1 = jnp.min(jnp.where(pm == p1, col, EP), axis=1, keepdims=True)
    pm2 = jnp.where(col == i1, neg, pm)
    p2 = jnp.max(pm2, axis=1, keepdims=True)
    i2 = jnp.min(jnp.where(pm2 == p2, col, EP), axis=1, keepdims=True)
    ps = p1 + p2
    i1_ref[...] = i1[:, 0]
    i2_ref[...] = i2[:, 0]
    p1_ref[...] = (p1 / ps)[:, 0]
    p2_ref[...] = (p2 / ps)[:, 0]

    oh = (col == i1).astype(jnp.float32) + (col == i2).astype(jnp.float32)
    cnt_blk = jnp.sum(oh, axis=0, keepdims=True)
    sp_blk = jnp.sum(probs, axis=0, keepdims=True)

    @pl.when(i == 0)
    def _():
        cnt_acc[...] = jnp.zeros_like(cnt_acc)
        sp_acc[...] = jnp.zeros_like(sp_acc)

    cnt_acc[...] += cnt_blk
    sp_acc[...] += sp_blk

    @pl.when(i == nsteps - 1)
    def _():
        aux_ref[...] = ((jnp.float32(E) / jnp.float32(S)) * jnp.sum(
            cnt_acc[...] * sp_acc[...])).reshape(1, 1)


def _router(x_flat, gw_pad):
    return pl.pallas_call(
        _router_body,
        grid=(S // TB,),
        in_specs=[
            pl.BlockSpec((TB, H), lambda i: (i, 0)),
            pl.BlockSpec((EP, H), lambda i: (0, 0)),
        ],
        out_specs=[
            pl.BlockSpec((TB,), lambda i: (i,)),
            pl.BlockSpec((TB,), lambda i: (i,)),
            pl.BlockSpec((TB,), lambda i: (i,)),
            pl.BlockSpec((TB,), lambda i: (i,)),
            pl.BlockSpec((1, 1), lambda i: (0, 0)),
        ],
        out_shape=[
            jax.ShapeDtypeStruct((S,), jnp.int32),
            jax.ShapeDtypeStruct((S,), jnp.int32),
            jax.ShapeDtypeStruct((S,), jnp.float32),
            jax.ShapeDtypeStruct((S,), jnp.float32),
            jax.ShapeDtypeStruct((1, 1), jnp.float32),
        ],
        scratch_shapes=[
            pltpu.VMEM((1, EP), jnp.float32),
            pltpu.VMEM((1, EP), jnp.float32),
        ],
    )(x_flat, gw_pad)


def _ffn_body(bmeta_ref, xg_ref, w1_ref, b1_ref, w2_ref, b2_ref, out_ref):
    i = pl.program_id(0)

    @pl.when(bmeta_ref[1, i] == 1)
    def _():
        xb = xg_ref[...].astype(jnp.bfloat16)
        h = lax.dot_general(
            xb, w1_ref[0].astype(jnp.bfloat16), (((1,), (1,)), ((), ())),
            preferred_element_type=jnp.float32)
        h = h + b1_ref[0]
        g = 0.5 * h * (1.0 + lax.erf(h * jnp.float32(0.7071067811865476)))
        y = lax.dot_general(
            g.astype(jnp.bfloat16), w2_ref[0].astype(jnp.bfloat16),
            (((1,), (1,)), ((), ())),
            preferred_element_type=jnp.float32)
        out_ref[...] = y + b2_ref[0]


def _ffn(bmeta, xg, fc1_w, fc1_b, fc2_w, fc2_b):
    grid_spec = pltpu.PrefetchScalarGridSpec(
        num_scalar_prefetch=1,
        grid=(NBLK,),
        in_specs=[
            pl.BlockSpec((BLK, H), lambda i, m: (i, 0)),
            pl.BlockSpec((1, F, H), lambda i, m: (m[0, i], 0, 0)),
            pl.BlockSpec((1, 1, F), lambda i, m: (m[0, i], 0, 0)),
            pl.BlockSpec((1, H, F), lambda i, m: (m[0, i], 0, 0)),
            pl.BlockSpec((1, 1, H), lambda i, m: (m[0, i], 0, 0)),
        ],
        out_specs=pl.BlockSpec((BLK, H), lambda i, m: (i, 0)),
    )
    return pl.pallas_call(
        _ffn_body,
        grid_spec=grid_spec,
        out_shape=jax.ShapeDtypeStruct((CAP, H), jnp.float32),
    )(bmeta, xg, fc1_w, fc1_b, fc2_w, fc2_b)


_TW2 = S // NW


def _sc_dispatch(x_flat, pos0, pos1):
    mesh = plsc.VectorSubcoreMesh(core_axis_name="c", subcore_axis_name="s")
    p2d0 = pos0.reshape(NW, _TW2)
    p2d1 = pos1.reshape(NW, _TW2)

    @functools.partial(
        pl.kernel, mesh=mesh,
        out_type=jax.ShapeDtypeStruct((CAP, H), jnp.float32),
        scratch_types=[
            pltpu.VMEM((_TW2,), jnp.int32),
            pltpu.VMEM((_TW2,), jnp.int32),
            pltpu.VMEM((_TW2, H), jnp.float32),
            pltpu.SemaphoreType.DMA,
            pltpu.SemaphoreType.DMA,
        ],
    )
    def k2(x_hbm, p0_hbm, p1_hbm, xg_hbm, i0_v, i1_v, rows_v, sem0, sem1):
        wid = lax.axis_index("s") * 2 + lax.axis_index("c")
        j0 = pltpu.async_copy(p0_hbm.at[wid], i0_v, sem0)
        j1 = pltpu.async_copy(p1_hbm.at[wid], i1_v, sem1)
        r = pltpu.async_copy(x_hbm.at[pl.ds(wid * _TW2, _TW2)], rows_v, sem0)
        j0.wait()
        j1.wait()
        r.wait()
        w0 = pltpu.async_copy(rows_v, xg_hbm.at[i0_v], sem0)
        w1 = pltpu.async_copy(rows_v, xg_hbm.at[i1_v], sem1)
        w0.wait()
        w1.wait()

    return k2(x_flat, p2d0, p2d1)


_TW = S // NW


def _sc_combine(yg, pos0, pos1, pb0):
    mesh = plsc.VectorSubcoreMesh(core_axis_name="c", subcore_axis_name="s")
    p2d0 = pos0.reshape(NW, _TW)
    p2d1 = pos1.reshape(NW, _TW)
    pb3d = pb0.reshape(NW, _TW, 16)

    @functools.partial(
        pl.kernel, mesh=mesh,
        out_type=jax.ShapeDtypeStruct((S, H), jnp.float32),
        scratch_types=[
            pltpu.VMEM((_TW,), jnp.int32),
            pltpu.VMEM((_TW,), jnp.int32),
            pltpu.VMEM((_TW, 16), jnp.float32),
            pltpu.VMEM((_TW, H), jnp.float32),
            pltpu.VMEM((_TW, H), jnp.float32),
            pltpu.SemaphoreType.DMA,
            pltpu.SemaphoreType.DMA,
        ],
    )
    def k4(yg_hbm, p0_hbm, p1_hbm, pb_hbm, o_hbm, i0_v, i1_v, pb_v, acc_v,
           tmp_v, sem, sem1):
        wid = lax.axis_index("s") * 2 + lax.axis_index("c")
        j0 = pltpu.async_copy(p0_hbm.at[wid], i0_v, sem)
        j1 = pltpu.async_copy(p1_hbm.at[wid], i1_v, sem1)
        jp = pltpu.async_copy(pb_hbm.at[wid], pb_v, sem)
        j0.wait()
        jp.wait()
        g0 = pltpu.async_copy(yg_hbm.at[i0_v], acc_v, sem)
        j1.wait()
        g1 = pltpu.async_copy(yg_hbm.at[i1_v], tmp_v, sem1)
        g0.wait()
        g1.wait()

        @pl.loop(0, _TW)
        def _(r):
            p = pb_v[r, :]
            for c in range(H // 16):
                sl = pl.ds(c * 16, 16)
                y1 = tmp_v[r, sl]
                acc_v[r, sl] = y1 + p * (acc_v[r, sl] - y1)

        pltpu.sync_copy(acc_v, o_hbm.at[pl.ds(wid * _TW, _TW)])

    return k4(yg, p2d0, p2d1, pb3d)


def kernel(x, gate_w, fc1_w, fc1_b, fc2_w, fc2_b):
    b, s, h = x.shape
    x_flat = x.reshape(s, h)
    gw_pad = jnp.zeros((EP, h), jnp.float32).at[:E].set(gate_w)

    i1, i2, p1n, p2n, aux = _router(x_flat, gw_pad)

    e_flat = jnp.stack([i1, i2], axis=1).reshape(-1)
    oh = (e_flat[:, None] == jnp.arange(E, dtype=jnp.int32)[None, :])
    csum = jnp.cumsum(oh.astype(jnp.int32), axis=0)
    counts = csum[-1]
    rank = jnp.take_along_axis(csum, e_flat[:, None], axis=1)[:, 0] - 1
    pc = ((counts + BLK - 1) // BLK) * BLK
    pstart = jnp.cumsum(pc) - pc
    pos = jnp.take(pstart, e_flat) + rank
    pos0, pos1 = pos[0::2], pos[1::2]
    pb0 = jnp.broadcast_to(p1n[:, None], (S, 16))
    nvb = jnp.sum(pc) // BLK
    bidx = jnp.arange(NBLK, dtype=jnp.int32)
    gend = (pstart + pc) // BLK
    bexp_raw = jnp.sum((bidx[:, None] >= gend[None, :]).astype(jnp.int32),
                       axis=1)
    bvalid = bidx < nvb
    last_exp = jnp.max(jnp.where(bvalid, bexp_raw, 0))
    bexp = jnp.where(bvalid, bexp_raw, last_exp)
    bnew = jnp.concatenate(
        [jnp.ones((1,), jnp.int32),
         (bexp[1:] != bexp[:-1]).astype(jnp.int32)])
    bmeta = jnp.stack([bexp, bvalid.astype(jnp.int32), bnew], axis=0)

    xg = _sc_dispatch(x_flat, pos0, pos1)
    yg = _ffn(bmeta, xg, fc1_w, fc1_b.reshape(E, 1, F),
              fc2_w, fc2_b.reshape(E, 1, H))
    out = _sc_combine(yg, pos0, pos1, pb0)
    return out.reshape(b, s, h), aux[0, 0]


# speedup vs baseline: 1.3376x; 1.0019x over previous
import functools

import jax
import jax.numpy as jnp
from jax import lax
from jax.experimental import pallas as pl
from jax.experimental.pallas import tpu as pltpu
from jax.experimental.pallas import tpu_sc as plsc

S = 2048
H = 768
E = 8
TOPK = 2
F = 3072
EP = 128
TB = 512
BLK = 128
NBLK = 40
CAP = NBLK * BLK
NW = 32


def _router_body(x_ref, gw_ref, i1_ref, i2_ref, p1_ref, p2_ref, aux_ref,
                 cnt_acc, sp_acc):
    i = pl.program_id(0)
    nsteps = pl.num_programs(0)
    logits = lax.dot_general(
        x_ref[...].astype(jnp.bfloat16), gw_ref[...].astype(jnp.bfloat16),
        (((1,), (1,)), ((), ())),
        preferred_element_type=jnp.float32)
    col = lax.broadcasted_iota(jnp.int32, (TB, EP), 1)
    neg = jnp.float32(-1e30)
    lg = jnp.where(col < E, logits, neg)
    m = jnp.max(lg, axis=1, keepdims=True)
    ex = jnp.where(col < E, jnp.exp(lg - m), 0.0)
    z = jnp.sum(ex, axis=1, keepdims=True)
    probs = ex / z
    pm = jnp.where(col < E, probs, neg)
    p1 = jnp.max(pm, axis=1, keepdims=True)
    i1 = jnp.min(jnp.where(pm == p1, col, EP), axis=1, keepdims=True)
    pm2 = jnp.where(col == i1, neg, pm)
    p2 = jnp.max(pm2, axis=1, keepdims=True)
    i2 = jnp.min(jnp.where(pm2 == p2, col, EP), axis=1, keepdims=True)
    ps = p1 + p2
    i1_ref[...] = i1[:, 0]
    i2_ref[...] = i2[:, 0]
    p1_ref[...] = (p1 / ps)[:, 0]
    p2_ref[...] = (p2 / ps)[:, 0]

    oh = (col == i1).astype(jnp.float32) + (col == i2).astype(jnp.float32)
    cnt_blk = jnp.sum(oh, axis=0, keepdims=True)
    sp_blk = jnp.sum(probs, axis=0, keepdims=True)

    @pl.when(i == 0)
    def _():
        cnt_acc[...] = jnp.zeros_like(cnt_acc)
        sp_acc[...] = jnp.zeros_like(sp_acc)

    cnt_acc[...] += cnt_blk
    sp_acc[...] += sp_blk

    @pl.when(i == nsteps - 1)
    def _():
        aux_ref[...] = ((jnp.float32(E) / jnp.float32(S)) * jnp.sum(
            cnt_acc[...] * sp_acc[...])).reshape(1, 1)


def _router(x_flat, gw_pad):
    return pl.pallas_call(
        _router_body,
        grid=(S // TB,),
        in_specs=[
            pl.BlockSpec((TB, H), lambda i: (i, 0)),
            pl.BlockSpec((EP, H), lambda i: (0, 0)),
        ],
        out_specs=[
            pl.BlockSpec((TB,), lambda i: (i,)),
            pl.BlockSpec((TB,), lambda i: (i,)),
            pl.BlockSpec((TB,), lambda i: (i,)),
            pl.BlockSpec((TB,), lambda i: (i,)),
            pl.BlockSpec((1, 1), lambda i: (0, 0)),
        ],
        out_shape=[
            jax.ShapeDtypeStruct((S,), jnp.int32),
            jax.ShapeDtypeStruct((S,), jnp.int32),
            jax.ShapeDtypeStruct((S,), jnp.float32),
            jax.ShapeDtypeStruct((S,), jnp.float32),
            jax.ShapeDtypeStruct((1, 1), jnp.float32),
        ],
        scratch_shapes=[
            pltpu.VMEM((1, EP), jnp.float32),
            pltpu.VMEM((1, EP), jnp.float32),
        ],
    )(x_flat, gw_pad)


def _ffn_body(bmeta_ref, xg_ref, w1_ref, b1_ref, w2_ref, b2_ref, out_ref):
    i = pl.program_id(0)

    @pl.when(bmeta_ref[1, i] == 1)
    def _():
        xb = xg_ref[...].astype(jnp.bfloat16)
        h = lax.dot_general(
            xb, w1_ref[0].astype(jnp.bfloat16), (((1,), (1,)), ((), ())),
            preferred_element_type=jnp.float32)
        h = h + b1_ref[0]
        g = 0.5 * h * (1.0 + lax.erf(h * jnp.float32(0.7071067811865476)))
        y = lax.dot_general(
            g.astype(jnp.bfloat16), w2_ref[0].astype(jnp.bfloat16),
            (((1,), (1,)), ((), ())),
            preferred_element_type=jnp.float32)
        out_ref[...] = y + b2_ref[0]


def _ffn(bmeta, xg, fc1_w, fc1_b, fc2_w, fc2_b):
    grid_spec = pltpu.PrefetchScalarGridSpec(
        num_scalar_prefetch=1,
        grid=(NBLK,),
        in_specs=[
            pl.BlockSpec((BLK, H), lambda i, m: (i, 0)),
            pl.BlockSpec((1, F, H), lambda i, m: (m[0, i], 0, 0)),
            pl.BlockSpec((1, 1, F), lambda i, m: (m[0, i], 0, 0)),
            pl.BlockSpec((1, H, F), lambda i, m: (m[0, i], 0, 0)),
            pl.BlockSpec((1, 1, H), lambda i, m: (m[0, i], 0, 0)),
        ],
        out_specs=pl.BlockSpec((BLK, H), lambda i, m: (i, 0)),
    )
    return pl.pallas_call(
        _ffn_body,
        grid_spec=grid_spec,
        out_shape=jax.ShapeDtypeStruct((CAP, H), jnp.float32),
    )(bmeta, xg, fc1_w, fc1_b, fc2_w, fc2_b)


_TW2 = S // NW


def _sc_dispatch(x_flat, pos0, pos1):
    mesh = plsc.VectorSubcoreMesh(core_axis_name="c", subcore_axis_name="s")
    p2d0 = pos0.reshape(NW, _TW2)
    p2d1 = pos1.reshape(NW, _TW2)

    @functools.partial(
        pl.kernel, mesh=mesh,
        out_type=jax.ShapeDtypeStruct((CAP, H), jnp.float32),
        scratch_types=[
            pltpu.VMEM((_TW2,), jnp.int32),
            pltpu.VMEM((_TW2,), jnp.int32),
            pltpu.VMEM((_TW2, H), jnp.float32),
            pltpu.SemaphoreType.DMA,
            pltpu.SemaphoreType.DMA,
        ],
    )
    def k2(x_hbm, p0_hbm, p1_hbm, xg_hbm, i0_v, i1_v, rows_v, sem0, sem1):
        wid = lax.axis_index("s") * 2 + lax.axis_index("c")
        j0 = pltpu.async_copy(p0_hbm.at[wid], i0_v, sem0)
        j1 = pltpu.async_copy(p1_hbm.at[wid], i1_v, sem1)
        r = pltpu.async_copy(x_hbm.at[pl.ds(wid * _TW2, _TW2)], rows_v, sem0)
        j0.wait()
        j1.wait()
        r.wait()
        w0 = pltpu.async_copy(rows_v, xg_hbm.at[i0_v], sem0)
        w1 = pltpu.async_copy(rows_v, xg_hbm.at[i1_v], sem1)
        w0.wait()
        w1.wait()

    return k2(x_flat, p2d0, p2d1)


_TW = S // NW


def _sc_combine(yg, pos0, pos1, pb0):
    mesh = plsc.VectorSubcoreMesh(core_axis_name="c", subcore_axis_name="s")
    p2d0 = pos0.reshape(NW, _TW)
    p2d1 = pos1.reshape(NW, _TW)
    pb3d = pb0.reshape(NW, _TW, 16)

    @functools.partial(
        pl.kernel, mesh=mesh,
        out_type=jax.ShapeDtypeStruct((S, H), jnp.float32),
        scratch_types=[
            pltpu.VMEM((_TW,), jnp.int32),
            pltpu.VMEM((_TW,), jnp.int32),
            pltpu.VMEM((_TW, 16), jnp.float32),
            pltpu.VMEM((_TW, H), jnp.float32),
            pltpu.VMEM((_TW, H), jnp.float32),
            pltpu.SemaphoreType.DMA,
            pltpu.SemaphoreType.DMA,
        ],
    )
    def k4(yg_hbm, p0_hbm, p1_hbm, pb_hbm, o_hbm, i0_v, i1_v, pb_v, acc_v,
           tmp_v, sem, sem1):
        wid = lax.axis_index("s") * 2 + lax.axis_index("c")
        j0 = pltpu.async_copy(p0_hbm.at[wid], i0_v, sem)
        j1 = pltpu.async_copy(p1_hbm.at[wid], i1_v, sem1)
        jp = pltpu.async_copy(pb_hbm.at[wid], pb_v, sem)
        j0.wait()
        jp.wait()
        g0 = pltpu.async_copy(yg_hbm.at[i0_v], acc_v, sem)
        j1.wait()
        g1 = pltpu.async_copy(yg_hbm.at[i1_v], tmp_v, sem1)
        g0.wait()
        g1.wait()

        @pl.loop(0, _TW)
        def _(r):
            p = pb_v[r, :]
            for c in range(H // 16):
                sl = pl.ds(c * 16, 16)
                y1 = tmp_v[r, sl]
                acc_v[r, sl] = y1 + p * (acc_v[r, sl] - y1)

        pltpu.sync_copy(acc_v, o_hbm.at[pl.ds(wid * _TW, _TW)])

    return k4(yg, p2d0, p2d1, pb3d)


def kernel(x, gate_w, fc1_w, fc1_b, fc2_w, fc2_b):
    b, s, h = x.shape
    x_flat = x.reshape(s, h)
    gw_pad = jnp.zeros((EP, h), jnp.float32).at[:E].set(gate_w)

    i1, i2, p1n, p2n, aux = _router(x_flat, gw_pad)

    e_flat = jnp.stack([i1, i2], axis=1).reshape(-1)
    oh = (e_flat[:, None] == jnp.arange(E, dtype=jnp.int32)[None, :])
    csum = jnp.cumsum(oh.astype(jnp.int32), axis=0)
    counts = csum[-1]
    rank = jnp.take_along_axis(csum, e_flat[:, None], axis=1)[:, 0] - 1
    pc = ((counts + BLK - 1) // BLK) * BLK
    pstart = jnp.cumsum(pc) - pc
    pos = jnp.take(pstart, e_flat) + rank
    pos0, pos1 = pos[0::2], pos[1::2]
    pb0 = jnp.broadcast_to(p1n[:, None], (S, 16))
    nvb = jnp.sum(pc) // BLK
    bidx = jnp.arange(NBLK, dtype=jnp.int32)
    gend = (pstart + pc) // BLK
    bexp_raw = jnp.sum((bidx[:, None] >= gend[None, :]).astype(jnp.int32),
                       axis=1)
    bvalid = bidx < nvb
    last_exp = jnp.max(jnp.where(bvalid, bexp_raw, 0))
    bexp = jnp.where(bvalid, bexp_raw, last_exp)
    bmeta = jnp.stack([bexp, bvalid.astype(jnp.int32)], axis=0)

    xg = _sc_dispatch(x_flat, pos0, pos1)
    yg = _ffn(bmeta, xg, fc1_w, fc1_b.reshape(E, 1, F),
              fc2_w, fc2_b.reshape(E, 1, H))
    out = _sc_combine(yg, pos0, pos1, pb0)
    return out.reshape(b, s, h), aux[0, 0]
